# baseline (speedup 1.0000x reference)
import functools

import jax
import jax.numpy as jnp
from jax import lax
from jax.experimental import pallas as pl
from jax.experimental.pallas import tpu as pltpu
from jax.experimental.pallas import tpu_sc as plsc

EMBED_DIM = 64
HIDDEN_DIM = 128
OUT_DIM = 32
BATCH = 16384
HIST = 200

NUM_WORKERS = 32
E_PER_W = BATCH // NUM_WORKERS
BLK_E = 64
NBLK = E_PER_W // BLK_E
INV_H = 1.0 / HIST

_mesh = plsc.VectorSubcoreMesh(core_axis_name="c", subcore_axis_name="s")


@functools.partial(
    pl.kernel,
    mesh=_mesh,
    out_type=jax.ShapeDtypeStruct((BATCH, EMBED_DIM), jnp.float32),
    scratch_types=[
        pltpu.VMEM((BLK_E, HIST), jnp.int32),
        pltpu.VMEM((BLK_E, HIST), jnp.int32),
        pltpu.VMEM((HIST, EMBED_DIM), jnp.float32),
        pltpu.VMEM((HIST, EMBED_DIM), jnp.float32),
        pltpu.VMEM((8, EMBED_DIM), jnp.float32),
        pltpu.SemaphoreType.DMA,
        pltpu.SemaphoreType.DMA,
    ],
    compiler_params=pltpu.CompilerParams(use_tc_tiling_on_sc=False),
)
def _pool(x_hbm, table_hbm, out_hbm, xb0, xb1, rb0, rb1, stage, sem0, sem1):
    wid = lax.axis_index("s") * 2 + lax.axis_index("c")
    row_base = wid * E_PER_W
    zero = jnp.zeros((16,), jnp.float32)
    xbufs = (xb0, xb1)
    rbufs = (rb0, rb1)
    sems = (sem0, sem1)

    def start(xbuf, le, rbuf, sem):
        pltpu.make_async_copy(
            table_hbm.at[xbuf.at[le, :]], rbuf, sem).start()

    def finish_and_accum(srow, rbuf, sem):
        pltpu.make_async_copy(table_hbm.at[xb0.at[0, :]], rbuf, sem).wait()

        def body(i, accs, rbuf=rbuf):
            a0, a1, a2, a3 = accs
            a0 = a0 + rbuf[i, pl.ds(0, 16)]
            a1 = a1 + rbuf[i, pl.ds(16, 16)]
            a2 = a2 + rbuf[i, pl.ds(32, 16)]
            a3 = a3 + rbuf[i, pl.ds(48, 16)]
            return (a0, a1, a2, a3)

        a0, a1, a2, a3 = lax.fori_loop(0, HIST, body,
                                       (zero, zero, zero, zero), unroll=8)
        stage[srow, pl.ds(0, 16)] = a0 * INV_H
        stage[srow, pl.ds(16, 16)] = a1 * INV_H
        stage[srow, pl.ds(32, 16)] = a2 * INV_H
        stage[srow, pl.ds(48, 16)] = a3 * INV_H

    pltpu.sync_copy(x_hbm.at[pl.ds(row_base, BLK_E), :], xb0)
    start(xb0, 0, rb0, sem0)

    for blk in range(NBLK):
        cur = xbufs[blk % 2]
        nxt = xbufs[(blk + 1) % 2]
        if blk < NBLK - 1:
            pltpu.sync_copy(
                x_hbm.at[pl.ds(row_base + (blk + 1) * BLK_E, BLK_E), :], nxt)

        def pair(p, carry, blk=blk, cur=cur):
            start(cur, 2 * p + 1, rb1, sem1)
            finish_and_accum((2 * p) % 8, rb0, sem0)

            @pl.when(p < BLK_E // 2 - 1)
            def _():
                start(cur, 2 * p + 2, rb0, sem0)

            finish_and_accum((2 * p + 1) % 8, rb1, sem1)

            @pl.when(p % 4 == 3)
            def _():
                pltpu.sync_copy(
                    stage,
                    out_hbm.at[pl.ds(
                        row_base + blk * BLK_E + (p // 4) * 8, 8)])

            return carry

        lax.fori_loop(0, BLK_E // 2, pair, 0)

        if blk < NBLK - 1:
            start(nxt, 0, rb0, sem0)


def _mlp_body(m_ref, w1_ref, b1_ref, w2_ref, b2_ref, o_ref):
    m = m_ref[...]
    h = lax.dot_general(m, w1_ref[...], (((1,), (0,)), ((), ())),
                        preferred_element_type=jnp.float32)
    h = jnp.maximum(h + b1_ref[...], 0.0)
    o = lax.dot_general(h, w2_ref[...], (((1,), (0,)), ((), ())),
                        preferred_element_type=jnp.float32)
    o = o + b2_ref[...]
    n = jnp.sqrt(jnp.sum(o * o, axis=1, keepdims=True) + 1e-08)
    o_ref[...] = o / n


def _mlp(m, W1, b1, W2, b2):
    blk = 2048
    grid = (BATCH // blk,)
    return pl.pallas_call(
        _mlp_body,
        grid=grid,
        in_specs=[
            pl.BlockSpec((blk, EMBED_DIM), lambda i: (i, 0)),
            pl.BlockSpec((EMBED_DIM, HIDDEN_DIM), lambda i: (0, 0)),
            pl.BlockSpec((1, HIDDEN_DIM), lambda i: (0, 0)),
            pl.BlockSpec((HIDDEN_DIM, OUT_DIM), lambda i: (0, 0)),
            pl.BlockSpec((1, OUT_DIM), lambda i: (0, 0)),
        ],
        out_specs=pl.BlockSpec((blk, OUT_DIM), lambda i: (i, 0)),
        out_shape=jax.ShapeDtypeStruct((BATCH, OUT_DIM), jnp.float32),
    )(m, W1, b1.reshape(1, -1), W2, b2.reshape(1, -1))


def kernel(x, table, W1, b1, W2, b2):
    m = _pool(x.astype(jnp.int32), table)
    return _mlp(m, W1, b1, W2, b2)
